# baseline (speedup 1.0000x reference)
import jax
import jax.numpy as jnp
from jax.experimental import pallas as pl
from jax.experimental.pallas import tpu as pltpu


def _body(tv_ref, w_ref, ti_ref, si_ref, li_ref, tt_ref, st_ref, lt_ref,
          b_ref, out_ref, c_ref):
    S = ti_ref.shape[0]

    @pl.when(pl.program_id(0) == 0)
    def _():
        def emb(ids_ref, table_ref):
            n = table_ref.shape[0]
            iota = jax.lax.broadcasted_iota(jnp.int32, (S, n), 1)
            oh = (ids_ref[...] == iota).astype(jnp.float32)
            return jax.lax.dot_general(
                oh, table_ref[...],
                dimension_numbers=(((1,), (0,)), ((), ())),
                preferred_element_type=jnp.float32)

        c_ref[...] = (emb(ti_ref, tt_ref) + emb(si_ref, st_ref)
                      + emb(li_ref, lt_ref) + b_ref[...])

    BB = tv_ref.shape[0]
    tv = tv_ref[...].reshape(BB, S, 1)
    out_ref[...] = tv * w_ref[...][None] + c_ref[...][None]


def kernel(token_values, W_val, b_val, type_table, side_table, slot_table,
           token_type_ids, token_side_ids, token_slot_ids):
    B, S = token_values.shape
    D = W_val.shape[0]

    ti = token_type_ids.reshape(S, 1)
    si = token_side_ids.reshape(S, 1)
    li = token_slot_ids.reshape(S, 1)
    w_row = W_val.reshape(1, D)
    b_row = b_val.reshape(1, D)

    BB = 128
    full = lambda shape: pl.BlockSpec(shape, lambda i: tuple(0 for _ in shape))
    return pl.pallas_call(
        _body,
        grid=(B // BB,),
        in_specs=[
            pl.BlockSpec((BB, S), lambda i: (i, 0)),
            full((1, D)),
            full((S, 1)), full((S, 1)), full((S, 1)),
            full(type_table.shape), full(side_table.shape), full(slot_table.shape),
            full((1, D)),
        ],
        out_specs=pl.BlockSpec((BB, S, D), lambda i: (i, 0, 0)),
        out_shape=jax.ShapeDtypeStruct((B, S, D), jnp.float32),
        scratch_shapes=[pltpu.VMEM((S, D), jnp.float32)],
        compiler_params=pltpu.CompilerParams(
            dimension_semantics=("arbitrary",)),
    )(token_values, w_row, ti, si, li, type_table, side_table, slot_table,
      b_row)
